# Initial kernel scaffold; baseline (speedup 1.0000x reference)
import jax
import jax.numpy as jnp
from jax.experimental import pallas as pl
from jax.experimental.pallas import tpu as pltpu

E = 64
H = 768
I = 1536
T = 2048
TOP_K = 2

TB = 256
IB = 512
NI = I // IB


def _router_body(x_ref, gw_ref, c_ref):
    xb = x_ref[...]
    gw = gw_ref[...]
    logits = jax.lax.dot_general(xb, gw, (((1,), (1,)), ((), ())),
                                 preferred_element_type=jnp.float32)
    m = jnp.max(logits, axis=1, keepdims=True)
    ex = jnp.exp(logits - m)
    probs = ex / jnp.sum(ex, axis=1, keepdims=True)

    ids = jax.lax.broadcasted_iota(jnp.int32, (TB, E), 1)
    i1 = jnp.argmax(probs, axis=1).astype(jnp.int32)[:, None]
    w1v = jnp.max(probs, axis=1, keepdims=True)
    oh1 = ids == i1
    probs2 = jnp.where(oh1, -jnp.inf, probs)
    i2 = jnp.argmax(probs2, axis=1).astype(jnp.int32)[:, None]
    w2v = jnp.max(probs2, axis=1, keepdims=True)
    oh2 = ids == i2
    c_ref[...] = jnp.where(oh1, w1v, 0.0) + jnp.where(oh2, w2v, 0.0)


def _expert_body(x_ref, c_ref, w1_ref, w3_ref, w2_ref, out_ref, yacc):
    i_blk = pl.program_id(1)
    xb = x_ref[...]
    w1b = w1_ref[0].astype(jnp.bfloat16)
    w3b = w3_ref[0].astype(jnp.bfloat16)
    w2b = w2_ref[0].astype(jnp.bfloat16)

    a = jax.lax.dot_general(xb, w1b, (((1,), (1,)), ((), ())),
                            preferred_element_type=jnp.float32)
    b = jax.lax.dot_general(xb, w3b, (((1,), (1,)), ((), ())),
                            preferred_element_type=jnp.float32)
    h = (a * jax.nn.sigmoid(a) * b).astype(jnp.bfloat16)
    y = jax.lax.dot_general(h, w2b, (((1,), (1,)), ((), ())),
                            preferred_element_type=jnp.float32)

    @pl.when(i_blk == 0)
    def _():
        yacc[...] = y

    @pl.when(i_blk > 0)
    def _():
        yacc[...] += y

    @pl.when(i_blk == NI - 1)
    def _():
        e = pl.program_id(0)
        contrib = c_ref[...] * yacc[...]

        @pl.when(e == 0)
        def _():
            out_ref[...] = contrib

        @pl.when(e > 0)
        def _():
            out_ref[...] += contrib


@jax.jit
def kernel(x, gate_w, w1, w2, w3):
    c = pl.pallas_call(
        _router_body,
        grid=(T // TB,),
        in_specs=[
            pl.BlockSpec((TB, H), lambda t: (t, 0)),
            pl.BlockSpec((E, H), lambda t: (0, 0)),
        ],
        out_specs=pl.BlockSpec((TB, E), lambda t: (t, 0)),
        out_shape=jax.ShapeDtypeStruct((T, E), jnp.float32),
    )(x, gate_w)

    x_bf = x.astype(jnp.bfloat16)

    out = pl.pallas_call(
        _expert_body,
        grid=(E, NI),
        in_specs=[
            pl.BlockSpec((T, H), lambda e, i: (0, 0)),
            pl.BlockSpec((T, 1), lambda e, i: (0, e)),
            pl.BlockSpec((1, IB, H), lambda e, i: (e, i, 0)),
            pl.BlockSpec((1, IB, H), lambda e, i: (e, i, 0)),
            pl.BlockSpec((1, H, IB), lambda e, i: (e, 0, i)),
        ],
        out_specs=pl.BlockSpec((T, H), lambda e, i: (0, 0)),
        out_shape=jax.ShapeDtypeStruct((T, H), jnp.float32),
        scratch_shapes=[pltpu.VMEM((T, H), jnp.float32)],
    )(x_bf, c, w1, w3, w2)
    return out


# speedup vs baseline: 2.0305x; 2.0305x over previous
import jax
import jax.numpy as jnp
from jax.experimental import pallas as pl
from jax.experimental.pallas import tpu as pltpu

E = 64
H = 768
I = 1536
T = 2048
TOP_K = 2

TB = 256
IB = 512
NI = I // IB


def _router_body(x_ref, gw_ref, c_ref):
    xb = x_ref[...]
    gw = gw_ref[...]
    logits = jax.lax.dot_general(xb, gw, (((1,), (1,)), ((), ())),
                                 preferred_element_type=jnp.float32)
    m = jnp.max(logits, axis=1, keepdims=True)
    ex = jnp.exp(logits - m)
    probs = ex / jnp.sum(ex, axis=1, keepdims=True)

    ids = jax.lax.broadcasted_iota(jnp.int32, (TB, E), 1)
    i1 = jnp.argmax(probs, axis=1).astype(jnp.int32)[:, None]
    w1v = jnp.max(probs, axis=1, keepdims=True)
    oh1 = ids == i1
    probs2 = jnp.where(oh1, -jnp.inf, probs)
    i2 = jnp.argmax(probs2, axis=1).astype(jnp.int32)[:, None]
    w2v = jnp.max(probs2, axis=1, keepdims=True)
    oh2 = ids == i2
    c_ref[...] = jnp.where(oh1, w1v, 0.0) + jnp.where(oh2, w2v, 0.0)


def _expert_body(x_ref, c_ref, w1_ref, w3_ref, w2_ref, out_ref, yacc):
    i_blk = pl.program_id(1)
    xb = x_ref[...]
    w1b = w1_ref[0].astype(jnp.bfloat16)
    w3b = w3_ref[0].astype(jnp.bfloat16)
    w2b = w2_ref[0].astype(jnp.bfloat16)

    a = jax.lax.dot_general(xb, w1b, (((1,), (1,)), ((), ())),
                            preferred_element_type=jnp.float32)
    b = jax.lax.dot_general(xb, w3b, (((1,), (1,)), ((), ())),
                            preferred_element_type=jnp.float32)
    h = (a * jax.nn.sigmoid(a) * b).astype(jnp.bfloat16)
    y = jax.lax.dot_general(h, w2b, (((1,), (1,)), ((), ())),
                            preferred_element_type=jnp.float32)

    @pl.when(i_blk == 0)
    def _():
        yacc[...] = y

    @pl.when(i_blk > 0)
    def _():
        yacc[...] += y

    @pl.when(i_blk == NI - 1)
    def _():
        e = pl.program_id(0)
        eids = jax.lax.broadcasted_iota(jnp.int32, (1, E), 1)
        c_col = jnp.sum(jnp.where(eids == e, c_ref[...], 0.0),
                        axis=1, keepdims=True)
        contrib = c_col * yacc[...]

        @pl.when(e == 0)
        def _():
            out_ref[...] = contrib

        @pl.when(e > 0)
        def _():
            out_ref[...] += contrib


@jax.jit
def kernel(x, gate_w, w1, w2, w3):
    c = pl.pallas_call(
        _router_body,
        grid=(T // TB,),
        in_specs=[
            pl.BlockSpec((TB, H), lambda t: (t, 0)),
            pl.BlockSpec((E, H), lambda t: (0, 0)),
        ],
        out_specs=pl.BlockSpec((TB, E), lambda t: (t, 0)),
        out_shape=jax.ShapeDtypeStruct((T, E), jnp.float32),
    )(x, gate_w)

    x_bf = x.astype(jnp.bfloat16)

    out = pl.pallas_call(
        _expert_body,
        grid=(E, NI),
        in_specs=[
            pl.BlockSpec((T, H), lambda e, i: (0, 0)),
            pl.BlockSpec((T, E), lambda e, i: (0, 0)),
            pl.BlockSpec((1, IB, H), lambda e, i: (e, i, 0)),
            pl.BlockSpec((1, IB, H), lambda e, i: (e, i, 0)),
            pl.BlockSpec((1, H, IB), lambda e, i: (e, 0, i)),
        ],
        out_specs=pl.BlockSpec((T, H), lambda e, i: (0, 0)),
        out_shape=jax.ShapeDtypeStruct((T, H), jnp.float32),
        scratch_shapes=[pltpu.VMEM((T, H), jnp.float32)],
    )(x_bf, c, w1, w3, w2)
    return out


# speedup vs baseline: 3.2027x; 1.5773x over previous
import jax
import jax.numpy as jnp
from jax.experimental import pallas as pl
from jax.experimental.pallas import tpu as pltpu

E = 64
H = 768
I = 1536
T = 2048
TOP_K = 2
TK = T * TOP_K

TB = 256
M = 64
NPT = 128
PAD = NPT * M


def _router_body(x_ref, gw_ref, rw_ref, se_ref):
    xb = x_ref[...]
    gw = gw_ref[...]
    logits = jax.lax.dot_general(xb, gw, (((1,), (1,)), ((), ())),
                                 preferred_element_type=jnp.float32)
    m = jnp.max(logits, axis=1, keepdims=True)
    ex = jnp.exp(logits - m)
    probs = ex / jnp.sum(ex, axis=1, keepdims=True)

    ids = jax.lax.broadcasted_iota(jnp.int32, (TB, E), 1)
    i1 = jnp.argmax(probs, axis=1).astype(jnp.int32)[:, None]
    w1v = jnp.max(probs, axis=1, keepdims=True)
    probs2 = jnp.where(ids == i1, -jnp.inf, probs)
    i2 = jnp.argmax(probs2, axis=1).astype(jnp.int32)[:, None]
    w2v = jnp.max(probs2, axis=1, keepdims=True)
    rw_ref[...] = jnp.concatenate([w1v, w2v], axis=1)
    se_ref[...] = jnp.concatenate([i1, i2], axis=1)


def _expert_body(e_of_ref, x_ref, st_ref, sw_ref, w1_ref, w3_ref, w2_ref,
                 out_ref):
    g = pl.program_id(0)
    stb = st_ref[0]
    swb = sw_ref[0]
    tok = jax.lax.broadcasted_iota(jnp.int32, (M, T), 1)
    oh = (stb == tok).astype(jnp.bfloat16)

    xg = jax.lax.dot_general(oh, x_ref[...], (((1,), (0,)), ((), ())),
                             preferred_element_type=jnp.float32)
    xg = xg.astype(jnp.bfloat16)

    w1b = w1_ref[0].astype(jnp.bfloat16)
    w3b = w3_ref[0].astype(jnp.bfloat16)
    w2b = w2_ref[0].astype(jnp.bfloat16)

    a = jax.lax.dot_general(xg, w1b, (((1,), (1,)), ((), ())),
                            preferred_element_type=jnp.float32)
    b = jax.lax.dot_general(xg, w3b, (((1,), (1,)), ((), ())),
                            preferred_element_type=jnp.float32)
    h = (a * jax.nn.sigmoid(a) * b).astype(jnp.bfloat16)
    y = jax.lax.dot_general(h, w2b, (((1,), (1,)), ((), ())),
                            preferred_element_type=jnp.float32)
    ys = (y * swb).astype(jnp.bfloat16)

    contrib = jax.lax.dot_general(oh, ys, (((0,), (0,)), ((), ())),
                                  preferred_element_type=jnp.float32)

    @pl.when(g == 0)
    def _():
        out_ref[...] = contrib

    @pl.when(g > 0)
    def _():
        out_ref[...] += contrib


@jax.jit
def kernel(x, gate_w, w1, w2, w3):
    rw, se = pl.pallas_call(
        _router_body,
        grid=(T // TB,),
        in_specs=[
            pl.BlockSpec((TB, H), lambda t: (t, 0)),
            pl.BlockSpec((E, H), lambda t: (0, 0)),
        ],
        out_specs=[
            pl.BlockSpec((TB, TOP_K), lambda t: (t, 0)),
            pl.BlockSpec((TB, TOP_K), lambda t: (t, 0)),
        ],
        out_shape=[
            jax.ShapeDtypeStruct((T, TOP_K), jnp.float32),
            jax.ShapeDtypeStruct((T, TOP_K), jnp.int32),
        ],
    )(x, gate_w)

    f = se.reshape(-1)
    rwf = rw.reshape(-1)
    counts = jnp.bincount(f, length=E)
    tiles = (counts + M - 1) // M
    csum_tiles = jnp.cumsum(tiles)
    tile_start = csum_tiles - tiles
    e_of = jnp.searchsorted(csum_tiles, jnp.arange(NPT), side="right")
    e_of = jnp.minimum(e_of, E - 1).astype(jnp.int32)

    order = jnp.argsort(f, stable=True)
    fs = f[order]
    grp_start = (jnp.cumsum(counts) - counts)[fs]
    rank = jnp.arange(TK, dtype=jnp.int32) - grp_start.astype(jnp.int32)
    pos = tile_start[fs].astype(jnp.int32) * M + rank

    st = jnp.zeros((PAD,), jnp.int32).at[pos].set(
        (order // TOP_K).astype(jnp.int32))
    sw = jnp.zeros((PAD,), jnp.float32).at[pos].set(rwf[order])
    st3 = st.reshape(NPT, M, 1)
    sw3 = sw.reshape(NPT, M, 1)

    x_bf = x.astype(jnp.bfloat16)

    out = pl.pallas_call(
        _expert_body,
        grid_spec=pltpu.PrefetchScalarGridSpec(
            num_scalar_prefetch=1,
            grid=(NPT,),
            in_specs=[
                pl.BlockSpec((T, H), lambda g, eo: (0, 0)),
                pl.BlockSpec((1, M, 1), lambda g, eo: (g, 0, 0)),
                pl.BlockSpec((1, M, 1), lambda g, eo: (g, 0, 0)),
                pl.BlockSpec((1, I, H), lambda g, eo: (eo[g], 0, 0)),
                pl.BlockSpec((1, I, H), lambda g, eo: (eo[g], 0, 0)),
                pl.BlockSpec((1, H, I), lambda g, eo: (eo[g], 0, 0)),
            ],
            out_specs=pl.BlockSpec((T, H), lambda g, eo: (0, 0)),
        ),
        out_shape=jax.ShapeDtypeStruct((T, H), jnp.float32),
    )(e_of, x_bf, st3, sw3, w1, w3, w2)
    return out
